# baseline (speedup 1.0000x reference)
import jax
import jax.numpy as jnp
from jax.experimental import pallas as pl
from jax.experimental.pallas import tpu as pltpu

_N = 100000
_D = 128
_G = 512
_T = 1000
_GRID = _N // _T
_TPAD = 1024
_NEG = -1e30


def _body(bi_ref, x_ref, wf1_ref, bf1_ref, wf2_ref, bf2_ref, ww1_ref,
          bw1_ref, ww2_ref, out_ref, m_ref, s_ref, acc_ref):
    i = pl.program_id(0)

    @pl.when(i == 0)
    def _init():
        m_ref[...] = jnp.full((_G, 1), _NEG, jnp.float32)
        s_ref[...] = jnp.zeros((_G, 1), jnp.float32)
        acc_ref[...] = jnp.zeros((_G, _D), jnp.float32)

    bf16 = jnp.bfloat16
    x = x_ref[...].astype(bf16)
    h = jnp.dot(x, wf1_ref[...].astype(bf16),
                preferred_element_type=jnp.float32) + bf1_ref[...]
    h = (h * jax.nn.sigmoid(h)).astype(bf16)
    p = jnp.dot(h, wf2_ref[...].astype(bf16),
                preferred_element_type=jnp.float32) + bf2_ref[...]
    hw = jnp.dot(x, ww1_ref[...].astype(bf16),
                 preferred_element_type=jnp.float32) + bw1_ref[...]
    hw = (hw * jax.nn.sigmoid(hw)).astype(bf16)
    w_row = jax.lax.dot_general(ww2_ref[...].astype(bf16), hw,
                                (((1,), (1,)), ((), ())),
                                preferred_element_type=jnp.float32)

    bi_row = bi_ref[0, :, :_T]
    gids = jax.lax.broadcasted_iota(jnp.int32, (_G, _T), 0)
    onehot = bi_row == gids

    tile_max = jnp.max(jnp.where(onehot, w_row, _NEG), axis=1, keepdims=True)
    m_old = m_ref[...]
    m_new = jnp.maximum(m_old, tile_max)
    scale = jnp.where(m_old > _NEG / 2, jnp.exp(m_old - m_new), 0.0)
    m_row = jnp.max(jnp.where(onehot, m_new, _NEG), axis=0, keepdims=True)
    e_row = jnp.exp(w_row - m_row)
    E = jnp.where(onehot, e_row, 0.0)
    s_ref[...] = s_ref[...] * scale + jnp.sum(E, axis=1, keepdims=True)
    acc_ref[...] = acc_ref[...] * scale + jnp.dot(
        E.astype(bf16), p.astype(bf16), preferred_element_type=jnp.float32)
    m_ref[...] = m_new

    @pl.when(i == _GRID - 1)
    def _finish():
        out_ref[...] = acc_ref[...] / (s_ref[...] + 1e-16)


def kernel(features, Wf1, bf1, Wf2, bf2, Ww1, bw1, Ww2, bw2, batch_index):
    del bw2
    bif = batch_index.reshape(_GRID, _T)
    bif = jnp.pad(bif, ((0, 0), (0, _TPAD - _T))).reshape(_GRID, 1, _TPAD)
    args = (bif, features,
            Wf1, bf1.reshape(1, _D), Wf2, bf2.reshape(1, _D),
            Ww1, bw1.reshape(1, _D), Ww2.reshape(1, _D))
    in_specs = [
        pl.BlockSpec((1, 1, _TPAD), lambda i: (i, 0, 0)),
        pl.BlockSpec((_T, _D), lambda i: (i, 0)),
        pl.BlockSpec((_D, _D), lambda i: (0, 0)),
        pl.BlockSpec((1, _D), lambda i: (0, 0)),
        pl.BlockSpec((_D, _D), lambda i: (0, 0)),
        pl.BlockSpec((1, _D), lambda i: (0, 0)),
        pl.BlockSpec((_D, _D), lambda i: (0, 0)),
        pl.BlockSpec((1, _D), lambda i: (0, 0)),
        pl.BlockSpec((1, _D), lambda i: (0, 0)),
    ]
    return pl.pallas_call(
        _body,
        grid=(_GRID,),
        in_specs=in_specs,
        out_specs=pl.BlockSpec((_G, _D), lambda i: (0, 0)),
        out_shape=jax.ShapeDtypeStruct((_G, _D), jnp.float32),
        scratch_shapes=[pltpu.VMEM((_G, 1), jnp.float32),
                        pltpu.VMEM((_G, 1), jnp.float32),
                        pltpu.VMEM((_G, _D), jnp.float32)],
        compiler_params=pltpu.CompilerParams(
            dimension_semantics=("arbitrary",)),
    )(*args)


# speedup vs baseline: 1.1716x; 1.1716x over previous
import jax
import jax.numpy as jnp
from jax.experimental import pallas as pl
from jax.experimental.pallas import tpu as pltpu

_N = 100000
_D = 128
_G = 512
_T = 1000
_GRID = _N // _T
_TPAD = 1024
_NEG = -1e30


def _body(bi_ref, x_ref, wf1_ref, bf1_ref, wf2_ref, bf2_ref, ww1_ref,
          bw1_ref, ww2_ref, out_ref, m_ref, s_ref, acc_ref):
    i = pl.program_id(0)

    @pl.when(i == 0)
    def _init():
        m_ref[0, 0] = _NEG
        s_ref[...] = jnp.zeros((_G, 1), jnp.float32)
        acc_ref[...] = jnp.zeros((_G, _D), jnp.float32)

    bf16 = jnp.bfloat16
    x = x_ref[...].astype(bf16)
    h = jnp.dot(x, wf1_ref[...].astype(bf16),
                preferred_element_type=jnp.float32) + bf1_ref[...]
    h = (h * jax.nn.sigmoid(h)).astype(bf16)
    p = jnp.dot(h, wf2_ref[...].astype(bf16),
                preferred_element_type=jnp.float32) + bf2_ref[...]
    hw = jnp.dot(x, ww1_ref[...].astype(bf16),
                 preferred_element_type=jnp.float32) + bw1_ref[...]
    hw = (hw * jax.nn.sigmoid(hw)).astype(bf16)
    w_row = jax.lax.dot_general(ww2_ref[...].astype(bf16), hw,
                                (((1,), (1,)), ((), ())),
                                preferred_element_type=jnp.float32)

    bi_row = bi_ref[0, :, :_T]
    gids = jax.lax.broadcasted_iota(jnp.int32, (_G, _T), 0)

    m_old = m_ref[0, 0]
    m_new = jnp.maximum(m_old, jnp.max(w_row))
    scale = jnp.exp(m_old - m_new)
    e_row = jnp.exp(jnp.maximum(w_row - m_new, -60.0))
    E = jnp.where(bi_row == gids, e_row, 0.0)
    s_ref[...] = s_ref[...] * scale + jnp.sum(E, axis=1, keepdims=True)
    acc_ref[...] = acc_ref[...] * scale + jnp.dot(
        E.astype(bf16), p.astype(bf16), preferred_element_type=jnp.float32)
    m_ref[0, 0] = m_new

    @pl.when(i == _GRID - 1)
    def _finish():
        out_ref[...] = acc_ref[...] / (s_ref[...] + 1e-16)


def kernel(features, Wf1, bf1, Wf2, bf2, Ww1, bw1, Ww2, bw2, batch_index):
    del bw2
    bif = batch_index.reshape(_GRID, _T)
    bif = jnp.pad(bif, ((0, 0), (0, _TPAD - _T))).reshape(_GRID, 1, _TPAD)
    args = (bif, features,
            Wf1, bf1.reshape(1, _D), Wf2, bf2.reshape(1, _D),
            Ww1, bw1.reshape(1, _D), Ww2.reshape(1, _D))
    in_specs = [
        pl.BlockSpec((1, 1, _TPAD), lambda i: (i, 0, 0)),
        pl.BlockSpec((_T, _D), lambda i: (i, 0)),
        pl.BlockSpec((_D, _D), lambda i: (0, 0)),
        pl.BlockSpec((1, _D), lambda i: (0, 0)),
        pl.BlockSpec((_D, _D), lambda i: (0, 0)),
        pl.BlockSpec((1, _D), lambda i: (0, 0)),
        pl.BlockSpec((_D, _D), lambda i: (0, 0)),
        pl.BlockSpec((1, _D), lambda i: (0, 0)),
        pl.BlockSpec((1, _D), lambda i: (0, 0)),
    ]
    return pl.pallas_call(
        _body,
        grid=(_GRID,),
        in_specs=in_specs,
        out_specs=pl.BlockSpec((_G, _D), lambda i: (0, 0)),
        out_shape=jax.ShapeDtypeStruct((_G, _D), jnp.float32),
        scratch_shapes=[pltpu.SMEM((1, 1), jnp.float32),
                        pltpu.VMEM((_G, 1), jnp.float32),
                        pltpu.VMEM((_G, _D), jnp.float32)],
        compiler_params=pltpu.CompilerParams(
            dimension_semantics=("arbitrary",)),
    )(*args)


# speedup vs baseline: 1.3011x; 1.1105x over previous
import jax
import jax.numpy as jnp
from jax.experimental import pallas as pl
from jax.experimental.pallas import tpu as pltpu

_N = 100000
_D = 128
_G = 512
_T = 2000
_GRID = _N // _T
_TPAD = 2048
_NEG = -1e30


def _body(bi_ref, x_ref, wf1_ref, bf1_ref, wf2_ref, bf2_ref, ww1_ref,
          bw1_ref, ww2_ref, out_ref, m_ref, s_ref, acc_ref):
    i = pl.program_id(0)

    @pl.when(i == 0)
    def _init():
        m_ref[0, 0] = _NEG
        s_ref[...] = jnp.zeros((_G, 1), jnp.float32)
        acc_ref[...] = jnp.zeros((_G, _D), jnp.float32)

    bf16 = jnp.bfloat16
    x = x_ref[...].astype(bf16)
    h = jnp.dot(x, wf1_ref[...].astype(bf16),
                preferred_element_type=jnp.float32) + bf1_ref[...]
    h = (h * jax.nn.sigmoid(h)).astype(bf16)
    p = jnp.dot(h, wf2_ref[...].astype(bf16),
                preferred_element_type=jnp.float32) + bf2_ref[...]
    hw = jnp.dot(x, ww1_ref[...].astype(bf16),
                 preferred_element_type=jnp.float32) + bw1_ref[...]
    hw = (hw * jax.nn.sigmoid(hw)).astype(bf16)
    w_row = jax.lax.dot_general(ww2_ref[...].astype(bf16), hw,
                                (((1,), (1,)), ((), ())),
                                preferred_element_type=jnp.float32)

    bi_row = bi_ref[0, :, :_T]
    gids = jax.lax.broadcasted_iota(jnp.int32, (_G, _T), 0)

    m_old = m_ref[0, 0]
    m_new = jnp.maximum(m_old, jnp.max(w_row))
    scale = jnp.exp(m_old - m_new)
    e_row = jnp.exp(jnp.maximum(w_row - m_new, -60.0))
    E = jnp.where(bi_row == gids, e_row, 0.0)
    s_ref[...] = s_ref[...] * scale + jnp.sum(E, axis=1, keepdims=True)
    acc_ref[...] = acc_ref[...] * scale + jnp.dot(
        E.astype(bf16), p.astype(bf16), preferred_element_type=jnp.float32)
    m_ref[0, 0] = m_new

    @pl.when(i == _GRID - 1)
    def _finish():
        out_ref[...] = acc_ref[...] / (s_ref[...] + 1e-16)


def kernel(features, Wf1, bf1, Wf2, bf2, Ww1, bw1, Ww2, bw2, batch_index):
    del bw2
    bif = batch_index.reshape(_GRID, _T)
    bif = jnp.pad(bif, ((0, 0), (0, _TPAD - _T))).reshape(_GRID, 1, _TPAD)
    args = (bif, features,
            Wf1, bf1.reshape(1, _D), Wf2, bf2.reshape(1, _D),
            Ww1, bw1.reshape(1, _D), Ww2.reshape(1, _D))
    in_specs = [
        pl.BlockSpec((1, 1, _TPAD), lambda i: (i, 0, 0)),
        pl.BlockSpec((_T, _D), lambda i: (i, 0)),
        pl.BlockSpec((_D, _D), lambda i: (0, 0)),
        pl.BlockSpec((1, _D), lambda i: (0, 0)),
        pl.BlockSpec((_D, _D), lambda i: (0, 0)),
        pl.BlockSpec((1, _D), lambda i: (0, 0)),
        pl.BlockSpec((_D, _D), lambda i: (0, 0)),
        pl.BlockSpec((1, _D), lambda i: (0, 0)),
        pl.BlockSpec((1, _D), lambda i: (0, 0)),
    ]
    return pl.pallas_call(
        _body,
        grid=(_GRID,),
        in_specs=in_specs,
        out_specs=pl.BlockSpec((_G, _D), lambda i: (0, 0)),
        out_shape=jax.ShapeDtypeStruct((_G, _D), jnp.float32),
        scratch_shapes=[pltpu.SMEM((1, 1), jnp.float32),
                        pltpu.VMEM((_G, 1), jnp.float32),
                        pltpu.VMEM((_G, _D), jnp.float32)],
        compiler_params=pltpu.CompilerParams(
            dimension_semantics=("arbitrary",)),
    )(*args)


# speedup vs baseline: 1.5568x; 1.1966x over previous
import jax
import jax.numpy as jnp
from jax.experimental import pallas as pl
from jax.experimental.pallas import tpu as pltpu

_N = 100000
_D = 128
_G = 512
_T = 2000
_GRID = _N // _T
_TPAD = 2048
_NEG = -1e30


def _body(bi_ref, x_ref, w1_ref, b1_ref, wf2_ref, bf2_ref, ww2_ref,
          out_ref, m_ref, s_ref, acc_ref):
    i = pl.program_id(0)

    @pl.when(i == 0)
    def _init():
        m_ref[0, 0] = _NEG
        s_ref[...] = jnp.zeros((_G, 1), jnp.float32)
        acc_ref[...] = jnp.zeros((_G, _D), jnp.float32)

    bf16 = jnp.bfloat16
    gids = jax.lax.broadcasted_iota(jnp.int32, (_G, _T), 0)

    x = x_ref[...].astype(bf16)
    hh = jnp.dot(x, w1_ref[...].astype(bf16),
                 preferred_element_type=jnp.float32) + b1_ref[...]
    hh = (hh * jax.nn.sigmoid(hh)).astype(bf16)
    p = jnp.dot(hh[:, :_D], wf2_ref[...].astype(bf16),
                preferred_element_type=jnp.float32) + bf2_ref[...]
    w_row = jax.lax.dot_general(ww2_ref[...].astype(bf16), hh[:, _D:],
                                (((1,), (1,)), ((), ())),
                                preferred_element_type=jnp.float32)

    bi_row = bi_ref[0, :, :_T]

    m_old = m_ref[0, 0]
    m_new = jnp.maximum(m_old, jnp.max(w_row))
    scale = jnp.exp(m_old - m_new)
    e_row = jnp.exp(jnp.maximum(w_row - m_new, -60.0))
    E = jnp.where(bi_row == gids, e_row, 0.0)
    s_ref[...] = s_ref[...] * scale + jnp.sum(E, axis=1, keepdims=True)
    acc_ref[...] = acc_ref[...] * scale + jnp.dot(
        E.astype(bf16), p.astype(bf16),
        preferred_element_type=jnp.float32)
    m_ref[0, 0] = m_new

    @pl.when(i == _GRID - 1)
    def _finish():
        out_ref[...] = acc_ref[...] / (s_ref[...] + 1e-16)


def kernel(features, Wf1, bf1, Wf2, bf2, Ww1, bw1, Ww2, bw2, batch_index):
    del bw2
    bif = batch_index.reshape(_GRID, _T)
    bif = jnp.pad(bif, ((0, 0), (0, _TPAD - _T))).reshape(_GRID, 1, _TPAD)
    w1 = jnp.concatenate([Wf1, Ww1], axis=1)
    b1 = jnp.concatenate([bf1, bw1]).reshape(1, 2 * _D)
    args = (bif, features, w1, b1,
            Wf2, bf2.reshape(1, _D), Ww2.reshape(1, _D))
    in_specs = [
        pl.BlockSpec((1, 1, _TPAD), lambda i: (i, 0, 0)),
        pl.BlockSpec((_T, _D), lambda i: (i, 0)),
        pl.BlockSpec((_D, 2 * _D), lambda i: (0, 0)),
        pl.BlockSpec((1, 2 * _D), lambda i: (0, 0)),
        pl.BlockSpec((_D, _D), lambda i: (0, 0)),
        pl.BlockSpec((1, _D), lambda i: (0, 0)),
        pl.BlockSpec((1, _D), lambda i: (0, 0)),
    ]
    return pl.pallas_call(
        _body,
        grid=(_GRID,),
        in_specs=in_specs,
        out_specs=pl.BlockSpec((_G, _D), lambda i: (0, 0)),
        out_shape=jax.ShapeDtypeStruct((_G, _D), jnp.float32),
        scratch_shapes=[pltpu.SMEM((1, 1), jnp.float32),
                        pltpu.VMEM((_G, 1), jnp.float32),
                        pltpu.VMEM((_G, _D), jnp.float32)],
        compiler_params=pltpu.CompilerParams(
            dimension_semantics=("arbitrary",)),
    )(*args)
